# baseline (speedup 1.0000x reference)
import jax
import jax.numpy as jnp
from jax.experimental import pallas as pl
from jax.experimental.pallas import tpu as pltpu

B, N, D, H, T = 8, 1024, 128, 32, 3
PB = 2


def _dot(a, b):
    return jax.lax.dot_general(
        a, b,
        (((a.ndim - 1,), (0,)), ((), ())),
        preferred_element_type=jnp.float32)


def _sigmoid(v):
    return 0.5 * jnp.tanh(0.5 * v) + 0.5


def _gru(a, x, wihT, bih, whhT, bhh):
    gi = _dot(a, wihT) + bih
    gh = _dot(x, whhT) + bhh
    r = _sigmoid(gi[:, :H] + gh[:, :H])
    z = _sigmoid(gi[:, H:2 * H] + gh[:, H:2 * H])
    n = jnp.tanh(gi[:, 2 * H:] + r * gh[:, 2 * H:])
    return (1.0 - z) * n + z * x


def _body_main(x_padded_ref, edges_ref, fc_wT_ref, fc_b_ref,
               W1_ref, wih1T_ref, whh1T_ref, bih1_ref, bhh1_ref,
               W2_ref, wih2T_ref, whh2T_ref, bih2_ref, bhh2_ref,
               out_ref, a_s):
    for bb in range(PB):
        out_ref[bb] = _dot(x_padded_ref[bb], fc_wT_ref[:]) + fc_b_ref[:]

    for (W_ref, wihT_ref, whhT_ref, bih_ref, bhh_ref) in (
            (W1_ref, wih1T_ref, whh1T_ref, bih1_ref, bhh1_ref),
            (W2_ref, wih2T_ref, whh2T_ref, bih2_ref, bhh2_ref)):
        for bb in range(PB):
            x = out_ref[bb]
            ai = _dot(edges_ref[bb, 0], _dot(x, W_ref[0]))
            ai += _dot(edges_ref[bb, 1], _dot(x, W_ref[1]))
            ai += _dot(edges_ref[bb, 2], _dot(x, W_ref[2]))
            a_s[bb] = ai
        for bb in range(PB):
            for blk in range(4):
                rows = slice(blk * (N // 4), (blk + 1) * (N // 4))
                out_ref[bb, rows] = _gru(a_s[bb, rows], out_ref[bb, rows],
                                         wihT_ref[:], bih_ref[:],
                                         whhT_ref[:], bhh_ref[:])


def _body_head(x2_ref, er5_ref, W3_ref, wih3T_ref, whh3T_ref,
               bih3_ref, bhh3_ref, out_wT_ref, out_b_ref, out_ref):
    for b in range(B):
        x = x2_ref[b]
        a3 = _dot(er5_ref[b, 0], _dot(x, W3_ref[0]))
        a3 += _dot(er5_ref[b, 1], _dot(x, W3_ref[1]))
        a3 += _dot(er5_ref[b, 2], _dot(x, W3_ref[2]))
        h = _gru(a3, x[5:6, :], wih3T_ref[:], bih3_ref[:],
                 whh3T_ref[:], bhh3_ref[:])
        logits = _dot(h, out_wT_ref[:]) + out_b_ref[:]
        mx = jnp.max(logits, axis=1, keepdims=True)
        lse = mx + jnp.log(jnp.sum(jnp.exp(logits - mx), axis=1,
                                   keepdims=True))
        out_ref[b] = logits - lse


@jax.jit
def kernel(x_padded, x_lengths, edges, fc_w, fc_b,
           W1, wih1, whh1, bih1, bhh1,
           W2, wih2, whh2, bih2, bhh2,
           W3, wih3, whh3, bih3, bhh3,
           out_w, out_b):
    del x_lengths

    def full(x):
        return pl.BlockSpec(x.shape, lambda b: (0,) * x.ndim)

    row2 = lambda v: v.reshape(1, -1)
    ins = (
        x_padded, edges,
        fc_w.T, row2(fc_b),
        W1, wih1.T, whh1.T, row2(bih1), row2(bhh1),
        W2, wih2.T, whh2.T, row2(bih2), row2(bhh2),
    )
    specs = [
        pl.BlockSpec((PB, N, D), lambda b: (b, 0, 0)),
        pl.BlockSpec((PB, T, N, N), lambda b: (b, 0, 0, 0)),
    ] + [full(x) for x in ins[2:]]

    x2 = pl.pallas_call(
        _body_main,
        grid=(B // PB,),
        in_specs=specs,
        out_specs=pl.BlockSpec((PB, N, H), lambda b: (b, 0, 0)),
        out_shape=jax.ShapeDtypeStruct((B, N, H), jnp.float32),
        scratch_shapes=[pltpu.VMEM((PB, N, H), jnp.float32)],
        compiler_params=pltpu.CompilerParams(
            dimension_semantics=("arbitrary",)),
    )(*ins)

    er5 = edges[:, :, 5:6, :]
    ins2 = (x2, er5, W3, wih3.T, whh3.T, row2(bih3), row2(bhh3),
            out_w.T, row2(out_b))
    out = pl.pallas_call(
        _body_head,
        in_specs=[pl.BlockSpec(x.shape, lambda x=x: (0,) * x.ndim)
                  for x in ins2],
        out_specs=pl.BlockSpec((B, 1, 5), lambda: (0, 0, 0)),
        out_shape=jax.ShapeDtypeStruct((B, 1, 5), jnp.float32),
    )(*ins2)
    return out.reshape(B, 5)


# speedup vs baseline: 1.0923x; 1.0923x over previous
import jax
import jax.numpy as jnp
from jax.experimental import pallas as pl
from jax.experimental.pallas import tpu as pltpu

B, N, D, H, T = 8, 1024, 128, 32, 3
PB = 2


def _dot(a, b):
    return jax.lax.dot_general(
        a, b,
        (((a.ndim - 1,), (0,)), ((), ())),
        preferred_element_type=jnp.float32)


def _sigmoid(v):
    return 0.5 * jnp.tanh(0.5 * v) + 0.5


def _gru(a, x, wihT, bih, whhT, bhh):
    gi = _dot(a, wihT) + bih
    gh = _dot(x, whhT) + bhh
    r = _sigmoid(gi[:, :H] + gh[:, :H])
    z = _sigmoid(gi[:, H:2 * H] + gh[:, H:2 * H])
    n = jnp.tanh(gi[:, 2 * H:] + r * gh[:, 2 * H:])
    return (1.0 - z) * n + z * x


def _body(x_padded_ref, edges_ref, fc_wT_ref, fc_b_ref,
          W1_ref, wih1T_ref, whh1T_ref, bih1_ref, bhh1_ref,
          W2_ref, wih2T_ref, whh2T_ref, bih2_ref, bhh2_ref,
          W3_ref, wih3T_ref, whh3T_ref, bih3_ref, bhh3_ref,
          out_wT_ref, out_b_ref, out_ref, x_s, a_s):
    for bb in range(PB):
        x_s[bb] = _dot(x_padded_ref[bb], fc_wT_ref[:]) + fc_b_ref[:]

    for (W_ref, wihT_ref, whhT_ref, bih_ref, bhh_ref) in (
            (W1_ref, wih1T_ref, whh1T_ref, bih1_ref, bhh1_ref),
            (W2_ref, wih2T_ref, whh2T_ref, bih2_ref, bhh2_ref)):
        for bb in range(PB):
            x = x_s[bb]
            ai = _dot(edges_ref[bb, 0], _dot(x, W_ref[0]))
            ai += _dot(edges_ref[bb, 1], _dot(x, W_ref[1]))
            ai += _dot(edges_ref[bb, 2], _dot(x, W_ref[2]))
            a_s[bb] = ai
        for bb in range(PB):
            for blk in range(4):
                rows = slice(blk * (N // 4), (blk + 1) * (N // 4))
                x_s[bb, rows] = _gru(a_s[bb, rows], x_s[bb, rows],
                                     wihT_ref[:], bih_ref[:],
                                     whhT_ref[:], bhh_ref[:])

    for bb in range(PB):
        x = x_s[bb]
        a3 = _dot(edges_ref[bb, 0, 5:6, :], _dot(x, W3_ref[0]))
        a3 += _dot(edges_ref[bb, 1, 5:6, :], _dot(x, W3_ref[1]))
        a3 += _dot(edges_ref[bb, 2, 5:6, :], _dot(x, W3_ref[2]))
        h = _gru(a3, x_s[bb, 5:6, :], wih3T_ref[:], bih3_ref[:],
                 whh3T_ref[:], bhh3_ref[:])

        logits = _dot(h, out_wT_ref[:]) + out_b_ref[:]
        mx = jnp.max(logits, axis=1, keepdims=True)
        lse = mx + jnp.log(jnp.sum(jnp.exp(logits - mx), axis=1,
                                   keepdims=True))
        out_ref[bb] = logits - lse


@jax.jit
def kernel(x_padded, x_lengths, edges, fc_w, fc_b,
           W1, wih1, whh1, bih1, bhh1,
           W2, wih2, whh2, bih2, bhh2,
           W3, wih3, whh3, bih3, bhh3,
           out_w, out_b):
    del x_lengths

    def full(x):
        return pl.BlockSpec(x.shape, lambda b: (0,) * x.ndim)

    row2 = lambda v: v.reshape(1, -1)
    ins = (
        x_padded, edges,
        fc_w.T, row2(fc_b),
        W1, wih1.T, whh1.T, row2(bih1), row2(bhh1),
        W2, wih2.T, whh2.T, row2(bih2), row2(bhh2),
        W3, wih3.T, whh3.T, row2(bih3), row2(bhh3),
        out_w.T, row2(out_b),
    )
    specs = [
        pl.BlockSpec((PB, N, D), lambda b: (b, 0, 0)),
        pl.BlockSpec((PB, T, N, N), lambda b: (b, 0, 0, 0)),
    ] + [full(x) for x in ins[2:]]

    out = pl.pallas_call(
        _body,
        grid=(B // PB,),
        in_specs=specs,
        out_specs=pl.BlockSpec((PB, 1, 5), lambda b: (b, 0, 0)),
        out_shape=jax.ShapeDtypeStruct((B, 1, 5), jnp.float32),
        scratch_shapes=[pltpu.VMEM((PB, N, H), jnp.float32),
                        pltpu.VMEM((PB, N, H), jnp.float32)],
        compiler_params=pltpu.CompilerParams(
            dimension_semantics=("arbitrary",)),
    )(*ins)
    return out.reshape(B, 5)
